# baseline (speedup 1.0000x reference)
import functools

import jax
import jax.numpy as jnp
from jax import lax
from jax.experimental import pallas as pl
from jax.experimental.pallas import tpu as pltpu
from jax.experimental.pallas import tpu_sc as plsc

_DIM = 2048
_NE = 8
_MAXIT = 3
_B = 4
_S = 4096

_S_SC = 1664
_S_TC = _S - _S_SC
_SBLK = 608
_NS_TC = _S_TC // _SBLK

_NW = 32
_WPB = _NW // _B
_RPW = _S_SC // _WPB
_TR = 8
_NT = _RPW // _TR
_NCHUNK = _DIM // 16


def _tc_norms_body(y_ref, yp_ref, an_ref, pd_ref, pa_ref):
    y = y_ref[0]
    d = y - yp_ref[0]
    a = y - an_ref[0]
    dn = jnp.sqrt(jnp.sum(d * d, axis=1, keepdims=True))
    an = jnp.sqrt(jnp.sum(a * a, axis=1, keepdims=True))
    pd_ref[0, 0, 0, 0] = jnp.sum(dn)
    pa_ref[0, 0, 0, 0] = jnp.sum(an)


def _sc_norms_body(y_hbm, yp_hbm, an_hbm, outd_hbm, outa_hbm,
                   ybuf0, ypbuf0, anbuf0, ybuf1, ypbuf1, anbuf1,
                   obufd, obufa, sem0, sem1):
    wid = lax.axis_index("c") * 16 + lax.axis_index("s")
    b_idx = wid // _WPB
    col0 = (wid % _WPB) * _RPW
    row0 = _S_TC + col0
    bufs = ((ybuf0, ypbuf0, anbuf0, sem0), (ybuf1, ypbuf1, anbuf1, sem1))
    lanes = lax.broadcasted_iota(jnp.int32, (16,), 0)
    zero = jnp.zeros((16,), jnp.float32)

    def issue(t, slot):
        yb, ypb, anb, sem = bufs[slot]
        row = row0 + t * _TR
        pltpu.async_copy(y_hbm.at[b_idx, pl.ds(row, _TR), :], yb, sem)
        pltpu.async_copy(yp_hbm.at[b_idx, pl.ds(row, _TR), :], ypb, sem)
        pltpu.async_copy(an_hbm.at[b_idx, pl.ds(row, _TR), :], anb, sem)

    def drain(t, slot):
        yb, ypb, anb, sem = bufs[slot]
        row = row0 + t * _TR
        pltpu.make_async_copy(y_hbm.at[b_idx, pl.ds(row, _TR), :], yb, sem).wait()
        pltpu.make_async_copy(yp_hbm.at[b_idx, pl.ds(row, _TR), :], ypb, sem).wait()
        pltpu.make_async_copy(an_hbm.at[b_idx, pl.ds(row, _TR), :], anb, sem).wait()

    issue(0, 0)

    def pair_body(i, carry):
        gvd, gva = carry
        for slot in (0, 1):
            t = 2 * i + slot
            drain(t, slot)

            @pl.when(t + 1 < _NT)
            def _():
                issue(t + 1, 1 - slot)

            yb, ypb, anb, _sem = bufs[slot]
            for r in range(_TR):
                def chunk(j, carry2):
                    accd, acca = carry2
                    yv = yb[r, pl.ds(j * 16, 16)]
                    d = yv - ypb[r, pl.ds(j * 16, 16)]
                    a = yv - anb[r, pl.ds(j * 16, 16)]
                    return accd + d * d, acca + a * a

                accd, acca = lax.fori_loop(0, _NCHUNK, chunk, (zero, zero),
                                           unroll=8)
                for k in (1, 2, 4, 8):
                    accd = accd + accd.at[lanes ^ k].get(mode="promise_in_bounds")
                    acca = acca + acca.at[lanes ^ k].get(mode="promise_in_bounds")
                lane = slot * _TR + r
                gvd = jnp.where(lanes == lane, accd, gvd)
                gva = jnp.where(lanes == lane, acca, gva)
            if slot == 1:
                obufd[pl.ds(i * 16, 16)] = gvd
                obufa[pl.ds(i * 16, 16)] = gva
        return gvd, gva

    lax.fori_loop(0, _NT // 2, pair_body, (zero, zero))
    g0 = wid * _RPW
    pltpu.sync_copy(obufd, outd_hbm.at[pl.ds(g0, _RPW)])
    pltpu.sync_copy(obufa, outa_hbm.at[pl.ds(g0, _RPW)])


_sc_norms = functools.partial(
    pl.kernel,
    out_type=[jax.ShapeDtypeStruct((_B * _S_SC,), jnp.float32)
              for _ in range(2)],
    mesh=plsc.VectorSubcoreMesh(core_axis_name="c", subcore_axis_name="s"),
    scratch_types=[
        pltpu.VMEM((_TR, _DIM), jnp.float32),
        pltpu.VMEM((_TR, _DIM), jnp.float32),
        pltpu.VMEM((_TR, _DIM), jnp.float32),
        pltpu.VMEM((_TR, _DIM), jnp.float32),
        pltpu.VMEM((_TR, _DIM), jnp.float32),
        pltpu.VMEM((_TR, _DIM), jnp.float32),
        pltpu.VMEM((_RPW,), jnp.float32),
        pltpu.VMEM((_RPW,), jnp.float32),
        pltpu.SemaphoreType.DMA,
        pltpu.SemaphoreType.DMA,
    ],
)(_sc_norms_body)


def _route_body(pd_ref, pa_ref, scd_ref, sca_ref, it_ref, w1_ref, b1_ref,
                w2_ref, idx_ref, out_ref):
    delta = (jnp.sum(pd_ref[...], axis=1, keepdims=True)
             + jnp.sum(jnp.sqrt(scd_ref[...]), axis=1, keepdims=True)) * (1.0 / _S)
    drift = (jnp.sum(pa_ref[...], axis=1, keepdims=True)
             + jnp.sum(jnp.sqrt(sca_ref[...]), axis=1, keepdims=True)) * (1.0 / _S)
    delta = delta / (jnp.mean(delta) + 1e-8)
    drift = drift / (jnp.mean(drift) + 1e-8)
    clamped = jnp.minimum(idx_ref[...], _MAXIT - 1)
    sel = (jax.lax.broadcasted_iota(jnp.int32, (1, _MAXIT), 1) == clamped
           ).astype(jnp.float32)
    emb = jax.lax.dot_general(sel, it_ref[...], (((1,), (0,)), ((), ())),
                              preferred_element_type=jnp.float32)
    emb4 = jnp.broadcast_to(emb, (_B, 16))
    x = jnp.concatenate([delta, drift, emb4], axis=1)
    h = jax.lax.dot_general(x, w1_ref[...], (((1,), (1,)), ((), ())),
                            preferred_element_type=jnp.float32) + b1_ref[...]
    xh = h[:, : _NE * 4]
    gate = h[:, _NE * 4:]
    h2 = (gate * jax.lax.logistic(gate)) * xh
    logits = jax.lax.dot_general(h2, w2_ref[...], (((1,), (1,)), ((), ())),
                                 preferred_element_type=jnp.float32)
    mx = jnp.max(logits, axis=1, keepdims=True)
    iota = jax.lax.broadcasted_iota(jnp.int32, (_B, _NE), 1)
    first = jnp.min(jnp.where(logits == mx, iota, _NE), axis=1, keepdims=True)
    onehot = (iota == first).astype(jnp.float32)
    out_ref[...] = jnp.zeros((8, 128), jnp.float32)
    out_ref[0:_B, 0:_NE] = onehot


def kernel(y, y_prev, linguistic_anchor, iter_table, W1, b1, W2, iter_idx):
    scd, sca = _sc_norms(y, y_prev, linguistic_anchor)
    scd = scd.reshape(_B, _S_SC)
    sca = sca.reshape(_B, _S_SC)
    pd, pa = pl.pallas_call(
        _tc_norms_body,
        grid=(_B, _NS_TC),
        in_specs=[pl.BlockSpec((1, _SBLK, _DIM), lambda b, s: (b, s, 0))
                  for _ in range(3)],
        out_specs=[pl.BlockSpec((1, 1, 1, 1), lambda b, s: (b, s, 0, 0),
                                memory_space=pltpu.SMEM)
                   for _ in range(2)],
        out_shape=[jax.ShapeDtypeStruct((_B, _NS_TC, 1, 1), jnp.float32)
                   for _ in range(2)],
    )(y, y_prev, linguistic_anchor)
    pd = pd.reshape(_B, _NS_TC)
    pa = pa.reshape(_B, _NS_TC)
    iidx = jnp.asarray(iter_idx, jnp.int32).reshape(1, 1)
    out = pl.pallas_call(
        _route_body,
        in_specs=[
            pl.BlockSpec((_B, _NS_TC), lambda: (0, 0)),
            pl.BlockSpec((_B, _NS_TC), lambda: (0, 0)),
            pl.BlockSpec((_B, _S_SC), lambda: (0, 0)),
            pl.BlockSpec((_B, _S_SC), lambda: (0, 0)),
            pl.BlockSpec((_MAXIT, 16), lambda: (0, 0)),
            pl.BlockSpec((64, 18), lambda: (0, 0)),
            pl.BlockSpec((1, 64), lambda: (0, 0)),
            pl.BlockSpec((_NE, 32), lambda: (0, 0)),
            pl.BlockSpec((1, 1), lambda: (0, 0)),
        ],
        out_specs=pl.BlockSpec((8, 128), lambda: (0, 0)),
        out_shape=jax.ShapeDtypeStruct((8, 128), jnp.float32),
    )(pd, pa, scd, sca, iter_table, W1, b1.reshape(1, 64), W2, iidx)
    return out[0:_B, 0:_NE]


# speedup vs baseline: 1.0487x; 1.0487x over previous
import functools

import jax
import jax.numpy as jnp
from jax import lax
from jax.experimental import pallas as pl
from jax.experimental.pallas import tpu as pltpu
from jax.experimental.pallas import tpu_sc as plsc

_DIM = 2048
_NE = 8
_MAXIT = 3
_B = 4
_S = 4096

_S_SC = 1664
_S_TC = _S - _S_SC
_SBLK = 608
_NS_TC = _S_TC // _SBLK

_NW = 32
_WPB = _NW // _B
_RPW = _S_SC // _WPB
_TR = 8
_NT = _RPW // _TR
_NCHUNK = _DIM // 16


def _tc_norms_body(y_ref, yp_ref, an_ref, pd_ref, pa_ref):
    y = y_ref[0]
    d = y - yp_ref[0]
    a = y - an_ref[0]
    dn = jnp.sqrt(jnp.sum(d * d, axis=1, keepdims=True))
    an = jnp.sqrt(jnp.sum(a * a, axis=1, keepdims=True))

    @pl.when(pl.program_id(1) == 0)
    def _():
        pd_ref[0, 0, 0, 0] = 0.0
        pa_ref[0, 0, 0, 0] = 0.0

    pd_ref[0, 0, 0, 0] += jnp.sum(dn)
    pa_ref[0, 0, 0, 0] += jnp.sum(an)


def _sc_norms_body(y_hbm, yp_hbm, an_hbm, outd_hbm, outa_hbm,
                   ybuf0, ypbuf0, anbuf0, ybuf1, ypbuf1, anbuf1,
                   obufd, obufa, sem0, sem1):
    wid = lax.axis_index("c") * 16 + lax.axis_index("s")
    b_idx = wid // _WPB
    col0 = (wid % _WPB) * _RPW
    row0 = _S_TC + col0
    bufs = ((ybuf0, ypbuf0, anbuf0, sem0), (ybuf1, ypbuf1, anbuf1, sem1))
    lanes = lax.broadcasted_iota(jnp.int32, (16,), 0)
    zero = jnp.zeros((16,), jnp.float32)

    def issue(t, slot):
        yb, ypb, anb, sem = bufs[slot]
        row = row0 + t * _TR
        pltpu.async_copy(y_hbm.at[b_idx, pl.ds(row, _TR), :], yb, sem)
        pltpu.async_copy(yp_hbm.at[b_idx, pl.ds(row, _TR), :], ypb, sem)
        pltpu.async_copy(an_hbm.at[b_idx, pl.ds(row, _TR), :], anb, sem)

    def drain(t, slot):
        yb, ypb, anb, sem = bufs[slot]
        row = row0 + t * _TR
        pltpu.make_async_copy(y_hbm.at[b_idx, pl.ds(row, _TR), :], yb, sem).wait()
        pltpu.make_async_copy(yp_hbm.at[b_idx, pl.ds(row, _TR), :], ypb, sem).wait()
        pltpu.make_async_copy(an_hbm.at[b_idx, pl.ds(row, _TR), :], anb, sem).wait()

    issue(0, 0)

    def pair_body(i, carry):
        gvd, gva = carry
        for slot in (0, 1):
            t = 2 * i + slot
            drain(t, slot)

            @pl.when(t + 1 < _NT)
            def _():
                issue(t + 1, 1 - slot)

            yb, ypb, anb, _sem = bufs[slot]
            for r in range(_TR):
                def chunk(j, carry2):
                    accd, acca = carry2
                    yv = yb[r, pl.ds(j * 16, 16)]
                    d = yv - ypb[r, pl.ds(j * 16, 16)]
                    a = yv - anb[r, pl.ds(j * 16, 16)]
                    return accd + d * d, acca + a * a

                accd, acca = lax.fori_loop(0, _NCHUNK, chunk, (zero, zero),
                                           unroll=8)
                for k in (1, 2, 4, 8):
                    accd = accd + accd.at[lanes ^ k].get(mode="promise_in_bounds")
                    acca = acca + acca.at[lanes ^ k].get(mode="promise_in_bounds")
                lane = slot * _TR + r
                gvd = jnp.where(lanes == lane, accd, gvd)
                gva = jnp.where(lanes == lane, acca, gva)
            if slot == 1:
                obufd[pl.ds(i * 16, 16)] = gvd
                obufa[pl.ds(i * 16, 16)] = gva
        return gvd, gva

    lax.fori_loop(0, _NT // 2, pair_body, (zero, zero))
    g0 = wid * _RPW
    pltpu.sync_copy(obufd, outd_hbm.at[pl.ds(g0, _RPW)])
    pltpu.sync_copy(obufa, outa_hbm.at[pl.ds(g0, _RPW)])


_sc_norms = functools.partial(
    pl.kernel,
    out_type=[jax.ShapeDtypeStruct((_B * _S_SC,), jnp.float32)
              for _ in range(2)],
    mesh=plsc.VectorSubcoreMesh(core_axis_name="c", subcore_axis_name="s"),
    scratch_types=[
        pltpu.VMEM((_TR, _DIM), jnp.float32),
        pltpu.VMEM((_TR, _DIM), jnp.float32),
        pltpu.VMEM((_TR, _DIM), jnp.float32),
        pltpu.VMEM((_TR, _DIM), jnp.float32),
        pltpu.VMEM((_TR, _DIM), jnp.float32),
        pltpu.VMEM((_TR, _DIM), jnp.float32),
        pltpu.VMEM((_RPW,), jnp.float32),
        pltpu.VMEM((_RPW,), jnp.float32),
        pltpu.SemaphoreType.DMA,
        pltpu.SemaphoreType.DMA,
    ],
)(_sc_norms_body)


def _route_body(pd_ref, pa_ref, scd_ref, sca_ref, it_ref, w1_ref, b1_ref,
                w2_ref, idx_ref, out_ref):
    nrow = scd_ref.shape[0] // _B
    rd = jnp.sum(jnp.sqrt(scd_ref[...]), axis=1, keepdims=True)
    ra = jnp.sum(jnp.sqrt(sca_ref[...]), axis=1, keepdims=True)
    seg = (jax.lax.broadcasted_iota(jnp.int32, (_B, _B * nrow), 1) // nrow
           == jax.lax.broadcasted_iota(jnp.int32, (_B, _B * nrow), 0)
           ).astype(jnp.float32)
    scdsum = jax.lax.dot_general(seg, rd, (((1,), (0,)), ((), ())),
                                 preferred_element_type=jnp.float32)
    scasum = jax.lax.dot_general(seg, ra, (((1,), (0,)), ((), ())),
                                 preferred_element_type=jnp.float32)
    bi = jax.lax.broadcasted_iota(jnp.int32, (_B, 1), 0)
    tcd = jnp.zeros((_B, 1), jnp.float32)
    tca = jnp.zeros((_B, 1), jnp.float32)
    for b in range(_B):
        tcd = jnp.where(bi == b, pd_ref[b, 0, 0, 0], tcd)
        tca = jnp.where(bi == b, pa_ref[b, 0, 0, 0], tca)
    delta = (tcd + scdsum) * (1.0 / _S)
    drift = (tca + scasum) * (1.0 / _S)
    delta = delta / (jnp.mean(delta) + 1e-8)
    drift = drift / (jnp.mean(drift) + 1e-8)
    clamped = jnp.minimum(idx_ref[...], _MAXIT - 1)
    sel = (jax.lax.broadcasted_iota(jnp.int32, (1, _MAXIT), 1) == clamped
           ).astype(jnp.float32)
    emb = jax.lax.dot_general(sel, it_ref[...], (((1,), (0,)), ((), ())),
                              preferred_element_type=jnp.float32)
    emb4 = jnp.broadcast_to(emb, (_B, 16))
    x = jnp.concatenate([delta, drift, emb4], axis=1)
    h = jax.lax.dot_general(x, w1_ref[...], (((1,), (1,)), ((), ())),
                            preferred_element_type=jnp.float32) + b1_ref[...]
    xh = h[:, : _NE * 4]
    gate = h[:, _NE * 4:]
    h2 = (gate * jax.lax.logistic(gate)) * xh
    logits = jax.lax.dot_general(h2, w2_ref[...], (((1,), (1,)), ((), ())),
                                 preferred_element_type=jnp.float32)
    mx = jnp.max(logits, axis=1, keepdims=True)
    iota = jax.lax.broadcasted_iota(jnp.int32, (_B, _NE), 1)
    first = jnp.min(jnp.where(logits == mx, iota, _NE), axis=1, keepdims=True)
    onehot = (iota == first).astype(jnp.float32)
    out_ref[...] = onehot


def kernel(y, y_prev, linguistic_anchor, iter_table, W1, b1, W2, iter_idx):
    scd, sca = _sc_norms(y, y_prev, linguistic_anchor)
    scd = scd.reshape(_B * _S_SC // 128, 128)
    sca = sca.reshape(_B * _S_SC // 128, 128)
    pd, pa = pl.pallas_call(
        _tc_norms_body,
        grid=(_B, _NS_TC),
        in_specs=[pl.BlockSpec((1, _SBLK, _DIM), lambda b, s: (b, s, 0))
                  for _ in range(3)],
        out_specs=[pl.BlockSpec((1, 1, 1, 1), lambda b, s: (b, 0, 0, 0),
                                memory_space=pltpu.SMEM)
                   for _ in range(2)],
        out_shape=[jax.ShapeDtypeStruct((_B, 1, 1, 1), jnp.float32)
                   for _ in range(2)],
    )(y, y_prev, linguistic_anchor)
    iidx = jnp.asarray(iter_idx, jnp.int32).reshape(1, 1)
    out = pl.pallas_call(
        _route_body,
        in_specs=[
            pl.BlockSpec((_B, 1, 1, 1), lambda: (0, 0, 0, 0),
                         memory_space=pltpu.SMEM),
            pl.BlockSpec((_B, 1, 1, 1), lambda: (0, 0, 0, 0),
                         memory_space=pltpu.SMEM),
            pl.BlockSpec((_B * _S_SC // 128, 128), lambda: (0, 0)),
            pl.BlockSpec((_B * _S_SC // 128, 128), lambda: (0, 0)),
            pl.BlockSpec((_MAXIT, 16), lambda: (0, 0)),
            pl.BlockSpec((64, 18), lambda: (0, 0)),
            pl.BlockSpec((1, 64), lambda: (0, 0)),
            pl.BlockSpec((_NE, 32), lambda: (0, 0)),
            pl.BlockSpec((1, 1), lambda: (0, 0)),
        ],
        out_specs=pl.BlockSpec((_B, _NE), lambda: (0, 0)),
        out_shape=jax.ShapeDtypeStruct((_B, _NE), jnp.float32),
    )(pd, pa, scd, sca, iter_table, W1, b1.reshape(1, 64), W2, iidx)
    return out


# speedup vs baseline: 1.0650x; 1.0155x over previous
import functools

import jax
import jax.numpy as jnp
from jax import lax
from jax.experimental import pallas as pl
from jax.experimental.pallas import tpu as pltpu
from jax.experimental.pallas import tpu_sc as plsc

_DIM = 2048
_NE = 8
_MAXIT = 3
_B = 4
_S = 4096

_S_SC = 512
_S_TC = _S - _S_SC
_SBLK = 512
_NS_TC = _S_TC // _SBLK

_NW = 32
_WPB = _NW // _B
_RPW = _S_SC // _WPB
_TR = 8
_NT = _RPW // _TR
_NCHUNK = _DIM // 16


def _tc_norms_body(y_ref, yp_ref, an_ref, pd_ref, pa_ref):
    y = y_ref[0]
    d = y - yp_ref[0]
    a = y - an_ref[0]
    dn = jnp.sqrt(jnp.sum(d * d, axis=1, keepdims=True))
    an = jnp.sqrt(jnp.sum(a * a, axis=1, keepdims=True))

    @pl.when(pl.program_id(1) == 0)
    def _():
        pd_ref[0, 0, 0, 0] = 0.0
        pa_ref[0, 0, 0, 0] = 0.0

    pd_ref[0, 0, 0, 0] += jnp.sum(dn)
    pa_ref[0, 0, 0, 0] += jnp.sum(an)


def _sc_norms_body(y_hbm, yp_hbm, an_hbm, outd_hbm, outa_hbm,
                   ybuf0, ypbuf0, anbuf0, ybuf1, ypbuf1, anbuf1,
                   obufd, obufa, sem0, sem1):
    wid = lax.axis_index("c") * 16 + lax.axis_index("s")
    b_idx = wid // _WPB
    col0 = (wid % _WPB) * _RPW
    row0 = _S_TC + col0
    bufs = ((ybuf0, ypbuf0, anbuf0, sem0), (ybuf1, ypbuf1, anbuf1, sem1))
    lanes = lax.broadcasted_iota(jnp.int32, (16,), 0)
    zero = jnp.zeros((16,), jnp.float32)

    def issue(t, slot):
        yb, ypb, anb, sem = bufs[slot]
        row = row0 + t * _TR
        pltpu.async_copy(y_hbm.at[b_idx, pl.ds(row, _TR), :], yb, sem)
        pltpu.async_copy(yp_hbm.at[b_idx, pl.ds(row, _TR), :], ypb, sem)
        pltpu.async_copy(an_hbm.at[b_idx, pl.ds(row, _TR), :], anb, sem)

    def drain(t, slot):
        yb, ypb, anb, sem = bufs[slot]
        row = row0 + t * _TR
        pltpu.make_async_copy(y_hbm.at[b_idx, pl.ds(row, _TR), :], yb, sem).wait()
        pltpu.make_async_copy(yp_hbm.at[b_idx, pl.ds(row, _TR), :], ypb, sem).wait()
        pltpu.make_async_copy(an_hbm.at[b_idx, pl.ds(row, _TR), :], anb, sem).wait()

    issue(0, 0)

    def pair_body(i, carry):
        gvd, gva = carry
        for slot in (0, 1):
            t = 2 * i + slot
            drain(t, slot)

            @pl.when(t + 1 < _NT)
            def _():
                issue(t + 1, 1 - slot)

            yb, ypb, anb, _sem = bufs[slot]
            for r in range(_TR):
                def chunk(j, carry2):
                    accd, acca = carry2
                    yv = yb[r, pl.ds(j * 16, 16)]
                    d = yv - ypb[r, pl.ds(j * 16, 16)]
                    a = yv - anb[r, pl.ds(j * 16, 16)]
                    return accd + d * d, acca + a * a

                accd, acca = lax.fori_loop(0, _NCHUNK, chunk, (zero, zero),
                                           unroll=8)
                for k in (1, 2, 4, 8):
                    accd = accd + accd.at[lanes ^ k].get(mode="promise_in_bounds")
                    acca = acca + acca.at[lanes ^ k].get(mode="promise_in_bounds")
                lane = slot * _TR + r
                gvd = jnp.where(lanes == lane, accd, gvd)
                gva = jnp.where(lanes == lane, acca, gva)
            if slot == 1:
                obufd[pl.ds(i * 16, 16)] = gvd
                obufa[pl.ds(i * 16, 16)] = gva
        return gvd, gva

    lax.fori_loop(0, _NT // 2, pair_body, (zero, zero))
    g0 = wid * _RPW
    pltpu.sync_copy(obufd, outd_hbm.at[pl.ds(g0, _RPW)])
    pltpu.sync_copy(obufa, outa_hbm.at[pl.ds(g0, _RPW)])


_sc_norms = functools.partial(
    pl.kernel,
    out_type=[jax.ShapeDtypeStruct((_B * _S_SC,), jnp.float32)
              for _ in range(2)],
    mesh=plsc.VectorSubcoreMesh(core_axis_name="c", subcore_axis_name="s"),
    scratch_types=[
        pltpu.VMEM((_TR, _DIM), jnp.float32),
        pltpu.VMEM((_TR, _DIM), jnp.float32),
        pltpu.VMEM((_TR, _DIM), jnp.float32),
        pltpu.VMEM((_TR, _DIM), jnp.float32),
        pltpu.VMEM((_TR, _DIM), jnp.float32),
        pltpu.VMEM((_TR, _DIM), jnp.float32),
        pltpu.VMEM((_RPW,), jnp.float32),
        pltpu.VMEM((_RPW,), jnp.float32),
        pltpu.SemaphoreType.DMA,
        pltpu.SemaphoreType.DMA,
    ],
)(_sc_norms_body)


def _route_body(pd_ref, pa_ref, scd_ref, sca_ref, it_ref, w1_ref, b1_ref,
                w2_ref, idx_ref, out_ref):
    nrow = scd_ref.shape[0] // _B
    rd = jnp.sum(jnp.sqrt(scd_ref[...]), axis=1, keepdims=True)
    ra = jnp.sum(jnp.sqrt(sca_ref[...]), axis=1, keepdims=True)
    seg = (jax.lax.broadcasted_iota(jnp.int32, (_B, _B * nrow), 1) // nrow
           == jax.lax.broadcasted_iota(jnp.int32, (_B, _B * nrow), 0)
           ).astype(jnp.float32)
    scdsum = jax.lax.dot_general(seg, rd, (((1,), (0,)), ((), ())),
                                 preferred_element_type=jnp.float32)
    scasum = jax.lax.dot_general(seg, ra, (((1,), (0,)), ((), ())),
                                 preferred_element_type=jnp.float32)
    bi = jax.lax.broadcasted_iota(jnp.int32, (_B, 1), 0)
    tcd = jnp.zeros((_B, 1), jnp.float32)
    tca = jnp.zeros((_B, 1), jnp.float32)
    for b in range(_B):
        tcd = jnp.where(bi == b, pd_ref[b, 0, 0, 0], tcd)
        tca = jnp.where(bi == b, pa_ref[b, 0, 0, 0], tca)
    delta = (tcd + scdsum) * (1.0 / _S)
    drift = (tca + scasum) * (1.0 / _S)
    delta = delta / (jnp.mean(delta) + 1e-8)
    drift = drift / (jnp.mean(drift) + 1e-8)
    clamped = jnp.minimum(idx_ref[...], _MAXIT - 1)
    sel = (jax.lax.broadcasted_iota(jnp.int32, (1, _MAXIT), 1) == clamped
           ).astype(jnp.float32)
    emb = jax.lax.dot_general(sel, it_ref[...], (((1,), (0,)), ((), ())),
                              preferred_element_type=jnp.float32)
    emb4 = jnp.broadcast_to(emb, (_B, 16))
    x = jnp.concatenate([delta, drift, emb4], axis=1)
    h = jax.lax.dot_general(x, w1_ref[...], (((1,), (1,)), ((), ())),
                            preferred_element_type=jnp.float32) + b1_ref[...]
    xh = h[:, : _NE * 4]
    gate = h[:, _NE * 4:]
    h2 = (gate * jax.lax.logistic(gate)) * xh
    logits = jax.lax.dot_general(h2, w2_ref[...], (((1,), (1,)), ((), ())),
                                 preferred_element_type=jnp.float32)
    mx = jnp.max(logits, axis=1, keepdims=True)
    iota = jax.lax.broadcasted_iota(jnp.int32, (_B, _NE), 1)
    first = jnp.min(jnp.where(logits == mx, iota, _NE), axis=1, keepdims=True)
    onehot = (iota == first).astype(jnp.float32)
    out_ref[...] = onehot


def kernel(y, y_prev, linguistic_anchor, iter_table, W1, b1, W2, iter_idx):
    scd, sca = _sc_norms(y, y_prev, linguistic_anchor)
    scd = scd.reshape(_B * _S_SC // 128, 128)
    sca = sca.reshape(_B * _S_SC // 128, 128)
    pd, pa = pl.pallas_call(
        _tc_norms_body,
        grid=(_B, _NS_TC),
        in_specs=[pl.BlockSpec((1, _SBLK, _DIM), lambda b, s: (b, s, 0))
                  for _ in range(3)],
        out_specs=[pl.BlockSpec((1, 1, 1, 1), lambda b, s: (b, 0, 0, 0),
                                memory_space=pltpu.SMEM)
                   for _ in range(2)],
        out_shape=[jax.ShapeDtypeStruct((_B, 1, 1, 1), jnp.float32)
                   for _ in range(2)],
    )(y, y_prev, linguistic_anchor)
    iidx = jnp.asarray(iter_idx, jnp.int32).reshape(1, 1)
    out = pl.pallas_call(
        _route_body,
        in_specs=[
            pl.BlockSpec((_B, 1, 1, 1), lambda: (0, 0, 0, 0),
                         memory_space=pltpu.SMEM),
            pl.BlockSpec((_B, 1, 1, 1), lambda: (0, 0, 0, 0),
                         memory_space=pltpu.SMEM),
            pl.BlockSpec((_B * _S_SC // 128, 128), lambda: (0, 0)),
            pl.BlockSpec((_B * _S_SC // 128, 128), lambda: (0, 0)),
            pl.BlockSpec((_MAXIT, 16), lambda: (0, 0)),
            pl.BlockSpec((64, 18), lambda: (0, 0)),
            pl.BlockSpec((1, 64), lambda: (0, 0)),
            pl.BlockSpec((_NE, 32), lambda: (0, 0)),
            pl.BlockSpec((1, 1), lambda: (0, 0)),
        ],
        out_specs=pl.BlockSpec((_B, _NE), lambda: (0, 0)),
        out_shape=jax.ShapeDtypeStruct((_B, _NE), jnp.float32),
    )(pd, pa, scd, sca, iter_table, W1, b1.reshape(1, 64), W2, iidx)
    return out


# speedup vs baseline: 1.2181x; 1.1438x over previous
import jax
import jax.numpy as jnp
from jax.experimental import pallas as pl
from jax.experimental.pallas import tpu as pltpu

_DIM = 2048
_NE = 8
_MAXIT = 3
_B = 4
_S = 4096
_SBLK = 512
_NS = _S // _SBLK


def _body(it_ref, w1_ref, b1_ref, w2_ref, idx_ref, y_ref, yp_ref, an_ref,
          out_ref, acc_ref):
    b = pl.program_id(0)
    s = pl.program_id(1)

    y = y_ref[0]
    d = y - yp_ref[0]
    a = y - an_ref[0]
    dn = jnp.sqrt(jnp.sum(d * d, axis=1, keepdims=True))
    an = jnp.sqrt(jnp.sum(a * a, axis=1, keepdims=True))

    @pl.when(s == 0)
    def _():
        acc_ref[0, b] = 0.0
        acc_ref[1, b] = 0.0

    acc_ref[0, b] += jnp.sum(dn)
    acc_ref[1, b] += jnp.sum(an)

    @pl.when((b == _B - 1) & (s == _NS - 1))
    def _():
        bi = jax.lax.broadcasted_iota(jnp.int32, (_B, 1), 0)
        delta = jnp.zeros((_B, 1), jnp.float32)
        drift = jnp.zeros((_B, 1), jnp.float32)
        for bb in range(_B):
            delta = jnp.where(bi == bb, acc_ref[0, bb], delta)
            drift = jnp.where(bi == bb, acc_ref[1, bb], drift)
        delta = delta * (1.0 / _S)
        drift = drift * (1.0 / _S)
        delta = delta / (jnp.mean(delta) + 1e-8)
        drift = drift / (jnp.mean(drift) + 1e-8)
        clamped = jnp.minimum(idx_ref[...], _MAXIT - 1)
        sel = (jax.lax.broadcasted_iota(jnp.int32, (1, _MAXIT), 1) == clamped
               ).astype(jnp.float32)
        emb = jax.lax.dot_general(sel, it_ref[...], (((1,), (0,)), ((), ())),
                                  preferred_element_type=jnp.float32)
        emb4 = jnp.broadcast_to(emb, (_B, 16))
        x = jnp.concatenate([delta, drift, emb4], axis=1)
        h = jax.lax.dot_general(x, w1_ref[...], (((1,), (1,)), ((), ())),
                                preferred_element_type=jnp.float32) + b1_ref[...]
        xh = h[:, : _NE * 4]
        gate = h[:, _NE * 4:]
        h2 = (gate * jax.lax.logistic(gate)) * xh
        logits = jax.lax.dot_general(h2, w2_ref[...], (((1,), (1,)), ((), ())),
                                     preferred_element_type=jnp.float32)
        mx = jnp.max(logits, axis=1, keepdims=True)
        iota = jax.lax.broadcasted_iota(jnp.int32, (_B, _NE), 1)
        first = jnp.min(jnp.where(logits == mx, iota, _NE), axis=1,
                        keepdims=True)
        out_ref[...] = (iota == first).astype(jnp.float32)


def kernel(y, y_prev, linguistic_anchor, iter_table, W1, b1, W2, iter_idx):
    iidx = jnp.asarray(iter_idx, jnp.int32).reshape(1, 1)
    return pl.pallas_call(
        _body,
        grid=(_B, _NS),
        in_specs=[
            pl.BlockSpec((_MAXIT, 16), lambda b, s: (0, 0)),
            pl.BlockSpec((64, 18), lambda b, s: (0, 0)),
            pl.BlockSpec((1, 64), lambda b, s: (0, 0)),
            pl.BlockSpec((_NE, 32), lambda b, s: (0, 0)),
            pl.BlockSpec((1, 1), lambda b, s: (0, 0)),
            pl.BlockSpec((1, _SBLK, _DIM), lambda b, s: (b, s, 0)),
            pl.BlockSpec((1, _SBLK, _DIM), lambda b, s: (b, s, 0)),
            pl.BlockSpec((1, _SBLK, _DIM), lambda b, s: (b, s, 0)),
        ],
        out_specs=pl.BlockSpec((_B, _NE), lambda b, s: (0, 0)),
        out_shape=jax.ShapeDtypeStruct((_B, _NE), jnp.float32),
        scratch_shapes=[pltpu.SMEM((2, _B), jnp.float32)],
    )(iter_table, W1, b1.reshape(1, 64), W2, iidx, y, y_prev,
      linguistic_anchor)
